# baseline (speedup 1.0000x reference)
import functools

import jax
import jax.numpy as jnp
from jax import lax
from jax.experimental import pallas as pl
from jax.experimental.pallas import tpu as pltpu
from jax.experimental.pallas import tpu_sc as plsc

N = 10000
E = 320000
D = 128

NC = 2
NS = 16
NW = NC * NS
RPW = 320
NPAD = NW * RPW

CH = 16000
NCHUNK = E // CH
GRP = CH // 16
GB = 32
NRING = 4
TRASH = CH + GB
CB = TRASH + 16

_INF = float("inf")
_PROBE_SKIP_GATHER = False
_PROBE_SKIP_ACCUM = False


def _tc_body(x_ref, wt_ref, wp_ref, bt_ref, bp_ref, a_ref, c_ref):
    xb = x_ref[...]
    dn = (((1,), (1,)), ((), ()))
    a = lax.dot_general(xb, wt_ref[...], dn, preferred_element_type=jnp.float32)
    p = lax.dot_general(xb, wp_ref[...], dn, preferred_element_type=jnp.float32)
    a_ref[...] = a
    c_ref[...] = a + p + bt_ref[...] + bp_ref[...]


def _tc_fn(xp, wt, wp, bt, bp):
    grid = 8
    blk = NPAD // grid
    return pl.pallas_call(
        _tc_body,
        grid=(grid,),
        in_specs=[
            pl.BlockSpec((blk, D), lambda i: (i, 0)),
            pl.BlockSpec((D, D), lambda i: (0, 0)),
            pl.BlockSpec((D, D), lambda i: (0, 0)),
            pl.BlockSpec((1, D), lambda i: (0, 0)),
            pl.BlockSpec((1, D), lambda i: (0, 0)),
        ],
        out_specs=[
            pl.BlockSpec((blk, D), lambda i: (i, 0)),
            pl.BlockSpec((blk, D), lambda i: (i, 0)),
        ],
        out_shape=[
            jax.ShapeDtypeStruct((NPAD, D), jnp.float32),
            jax.ShapeDtypeStruct((NPAD, D), jnp.float32),
        ],
    )(xp, wt, wp, bt, bp)


def _sc_body(a_hbm, c_hbm, src_hbm, dst_hbm, out_hbm,
             acc, gb0, gb1, gb2, gb3, dst_v, src_v, scmp, lcmp,
             sm0, sm1, sm2, sm3, semd, semsrc):
    gbufs = (gb0, gb1, gb2, gb3)
    sems = (sm0, sm1, sm2, sm3)
    wid = lax.axis_index("s") * NC + lax.axis_index("c")
    lo = wid * RPW
    hi = lo + RPW

    inf_vec = jnp.full((16,), _INF, jnp.float32)
    zero_vec = jnp.zeros((16,), jnp.int32)
    ones16 = jnp.full((16,), 1, jnp.int32)
    dummy_vec = jnp.full((16,), RPW, jnp.int32)
    zeros16 = jnp.zeros((16,), jnp.int32)

    def init_row(r, _):
        for f in range(8):
            acc[r, pl.ds(f * 16, 16)] = inf_vec
        return 0
    lax.fori_loop(0, RPW + 1, init_row, 0)

    pltpu.async_copy(dst_hbm.at[pl.ds(0, CH)], dst_v, semd)
    pltpu.async_copy(src_hbm.at[pl.ds(0, CH)], src_v, semsrc)

    def chunk_body(ci, _):
        pltpu.make_async_copy(dst_hbm.at[pl.ds(0, CH)], dst_v, semd).wait()
        pltpu.make_async_copy(src_hbm.at[pl.ds(0, CH)], src_v, semsrc).wait()

        def scan_body(g, cur):
            off = g * 16
            dvec = dst_v[pl.ds(off, 16)]
            mask = jnp.logical_and(dvec >= lo, dvec < hi)
            cnt = plsc.all_reduce_population_count(mask)[0]

            @pl.when(cnt > 0)
            def _():
                svec = src_v[pl.ds(off, 16)]
                cs = plsc.cumsum(jnp.where(mask, ones16, zeros16))
                lane = lax.iota(jnp.int32, 16)
                pos = jnp.where(mask, cur + cs - 1, TRASH + lane)
                plsc.store_scatter(scmp, [pos], svec)
                plsc.store_scatter(lcmp, [pos], dvec - lo)
            return cur + cnt
        n = lax.fori_loop(0, GRP, scan_body, jnp.int32(0))

        @pl.when(ci + 1 < NCHUNK)
        def _():
            nbase = (ci + 1) * CH
            pltpu.async_copy(dst_hbm.at[pl.ds(nbase, CH)], dst_v, semd)
            pltpu.async_copy(src_hbm.at[pl.ds(nbase, CH)], src_v, semsrc)

        def pad_body(t, _):
            scmp[pl.ds(n + t * 16, 16)] = zero_vec
            lcmp[pl.ds(n + t * 16, 16)] = dummy_vec
            return 0
        lax.fori_loop(0, GB // 16, pad_body, 0)

        ng = (n + GB - 1) // GB

        def accum_block(g, buf):
            base = g * GB

            def edge_body(j, _):
                r = lcmp[pl.ds(base + j, 16)][0]
                for f in range(8):
                    sl = pl.ds(f * 16, 16)
                    acc[r, sl] = jnp.minimum(acc[r, sl], buf[j, sl])
                return 0
            if not _PROBE_SKIP_ACCUM:
                lax.fori_loop(0, GB, edge_body, 0)

        def pipe_body(g, _):
            for s_ in range(NRING):
                @pl.when(jnp.logical_and(g < ng, g % NRING == s_))
                def _(s_=s_):
                    pltpu.async_copy(a_hbm.at[scmp.at[pl.ds(g * GB, GB)]],
                                     gbufs[s_], sems[s_])

            for s_ in range(NRING):
                @pl.when(jnp.logical_and(g >= NRING - 1,
                                         (g - (NRING - 1)) % NRING == s_))
                def _(s_=s_):
                    pltpu.make_async_copy(a_hbm.at[pl.ds(0, GB)],
                                          gbufs[s_], sems[s_]).wait()
                    accum_block(g - (NRING - 1), gbufs[s_])
            return 0
        if not _PROBE_SKIP_GATHER:
            lax.fori_loop(0, ng + NRING - 1, pipe_body, 0)
        return 0

    lax.fori_loop(0, NCHUNK, chunk_body, 0)

    for k in range(RPW // GB):
        rbase = lo + k * GB
        pltpu.sync_copy(c_hbm.at[pl.ds(rbase, GB)], gb0)

        def comb_body(r, _):
            ra = k * GB + r
            for f in range(8):
                sl = pl.ds(f * 16, 16)
                a = acc[ra, sl]
                cv = gb0[r, sl]
                gb0[r, sl] = jnp.where(a < jnp.float32(_INF), cv - a,
                                       jnp.float32(0.0))
            return 0
        lax.fori_loop(0, GB, comb_body, 0)
        pltpu.sync_copy(gb0, out_hbm.at[pl.ds(rbase, GB)])


_sc_fn = pl.kernel(
    _sc_body,
    out_type=jax.ShapeDtypeStruct((NPAD, D), jnp.float32),
    mesh=plsc.VectorSubcoreMesh(core_axis_name="c", subcore_axis_name="s"),
    scratch_types=[
        pltpu.VMEM((RPW + 1, D), jnp.float32),
        pltpu.VMEM((GB, D), jnp.float32),
        pltpu.VMEM((GB, D), jnp.float32),
        pltpu.VMEM((GB, D), jnp.float32),
        pltpu.VMEM((GB, D), jnp.float32),
        pltpu.VMEM((CH,), jnp.int32),
        pltpu.VMEM((CH,), jnp.int32),
        pltpu.VMEM((CB,), jnp.int32),
        pltpu.VMEM((CB,), jnp.int32),
        pltpu.SemaphoreType.DMA,
        pltpu.SemaphoreType.DMA,
        pltpu.SemaphoreType.DMA,
        pltpu.SemaphoreType.DMA,
        pltpu.SemaphoreType.DMA,
        pltpu.SemaphoreType.DMA,
    ],
    compiler_params=pltpu.CompilerParams(needs_layout_passes=False),
)


@jax.jit
def kernel(x, edge_index, W_theta, b_theta, W_phi, b_phi):
    src = edge_index[0]
    dst = edge_index[1]
    xp = jnp.pad(x, ((0, NPAD - N), (0, 0)))
    a, c = _tc_fn(xp, W_theta, W_phi,
                  b_theta.reshape(1, D), b_phi.reshape(1, D))
    out = _sc_fn(a, c, src, dst)
    return out[:N]


# speedup vs baseline: 1.4459x; 1.4459x over previous
import functools

import jax
import jax.numpy as jnp
from jax import lax
from jax.experimental import pallas as pl
from jax.experimental.pallas import tpu as pltpu
from jax.experimental.pallas import tpu_sc as plsc

N = 10000
E = 320000
D = 128

NC = 2
NS = 16
NW = NC * NS
RPW = 320
NPAD = NW * RPW

CH = 16000
NCHUNK = E // CH
GRP = CH // 16
GB = 32
NRING = 4
TRASH = CH + GB
CB = TRASH + 16

_INF = float("inf")
_PROBE_SKIP_GATHER = False
_PROBE_SKIP_ACCUM = False


def _tc_body(x_ref, wt_ref, wp_ref, bt_ref, bp_ref, a_ref, c_ref):
    xb = x_ref[...]
    dn = (((1,), (1,)), ((), ()))
    a = lax.dot_general(xb, wt_ref[...], dn, preferred_element_type=jnp.float32)
    p = lax.dot_general(xb, wp_ref[...], dn, preferred_element_type=jnp.float32)
    a_ref[...] = a
    c_ref[...] = a + p + bt_ref[...] + bp_ref[...]


def _tc_fn(xp, wt, wp, bt, bp):
    grid = 8
    blk = NPAD // grid
    return pl.pallas_call(
        _tc_body,
        grid=(grid,),
        in_specs=[
            pl.BlockSpec((blk, D), lambda i: (i, 0)),
            pl.BlockSpec((D, D), lambda i: (0, 0)),
            pl.BlockSpec((D, D), lambda i: (0, 0)),
            pl.BlockSpec((1, D), lambda i: (0, 0)),
            pl.BlockSpec((1, D), lambda i: (0, 0)),
        ],
        out_specs=[
            pl.BlockSpec((blk, D), lambda i: (i, 0)),
            pl.BlockSpec((blk, D), lambda i: (i, 0)),
        ],
        out_shape=[
            jax.ShapeDtypeStruct((NPAD, D), jnp.float32),
            jax.ShapeDtypeStruct((NPAD, D), jnp.float32),
        ],
    )(xp, wt, wp, bt, bp)


def _sc_body(a_hbm, c_hbm, src_hbm, dst_hbm, out_hbm,
             acc, gb0, gb1, gb2, gb3, dst_v, src_v, scmp, lcmp,
             sm0, sm1, sm2, sm3, semd, semsrc):
    gbufs = (gb0, gb1, gb2, gb3)
    sems = (sm0, sm1, sm2, sm3)
    wid = lax.axis_index("s") * NC + lax.axis_index("c")
    lo = wid * RPW
    hi = lo + RPW

    inf_vec = jnp.full((16,), _INF, jnp.float32)
    zero_vec = jnp.zeros((16,), jnp.int32)
    ones16 = jnp.full((16,), 1, jnp.int32)
    dummy_vec = jnp.full((16,), RPW, jnp.int32)
    zeros16 = jnp.zeros((16,), jnp.int32)

    def init_row(r, _):
        for f in range(8):
            acc[r, pl.ds(f * 16, 16)] = inf_vec
        return 0
    lax.fori_loop(0, RPW + 1, init_row, 0)

    pltpu.async_copy(dst_hbm.at[pl.ds(0, CH)], dst_v, semd)
    pltpu.async_copy(src_hbm.at[pl.ds(0, CH)], src_v, semsrc)

    def chunk_body(ci, _):
        pltpu.make_async_copy(dst_hbm.at[pl.ds(0, CH)], dst_v, semd).wait()
        pltpu.make_async_copy(src_hbm.at[pl.ds(0, CH)], src_v, semsrc).wait()

        def scan_body(g, cur):
            off = g * 16
            dvec = dst_v[pl.ds(off, 16)]
            svec = src_v[pl.ds(off, 16)]
            mask = jnp.logical_and(dvec >= lo, dvec < hi)
            cs = plsc.cumsum(jnp.where(mask, ones16, zeros16))
            lane = lax.iota(jnp.int32, 16)
            pos = jnp.where(mask, cur + cs - 1, TRASH + lane)
            plsc.store_scatter(scmp, [pos], svec)
            plsc.store_scatter(lcmp, [pos], dvec - lo)
            return cur + cs[15]
        n = lax.fori_loop(0, GRP, scan_body, jnp.int32(0))

        @pl.when(ci + 1 < NCHUNK)
        def _():
            nbase = (ci + 1) * CH
            pltpu.async_copy(dst_hbm.at[pl.ds(nbase, CH)], dst_v, semd)
            pltpu.async_copy(src_hbm.at[pl.ds(nbase, CH)], src_v, semsrc)

        def pad_body(t, _):
            scmp[pl.ds(n + t * 16, 16)] = zero_vec
            lcmp[pl.ds(n + t * 16, 16)] = dummy_vec
            return 0
        lax.fori_loop(0, GB // 16, pad_body, 0)

        ng = (n + GB - 1) // GB

        def accum_block(g, buf):
            base = g * GB

            def batch_body(t, _):
                lvec = lcmp[pl.ds(base + t * 16, 16)]
                for i_ in range(16):
                    r = lvec[i_]
                    j = t * 16 + i_
                    for f in range(8):
                        sl = pl.ds(f * 16, 16)
                        acc[r, sl] = jnp.minimum(acc[r, sl], buf[j, sl])
                return 0
            if not _PROBE_SKIP_ACCUM:
                lax.fori_loop(0, GB // 16, batch_body, 0)

        def pipe_body(g, _):
            for s_ in range(NRING):
                @pl.when(jnp.logical_and(g < ng, g % NRING == s_))
                def _(s_=s_):
                    pltpu.async_copy(a_hbm.at[scmp.at[pl.ds(g * GB, GB)]],
                                     gbufs[s_], sems[s_])

            for s_ in range(NRING):
                @pl.when(jnp.logical_and(g >= NRING - 1,
                                         (g - (NRING - 1)) % NRING == s_))
                def _(s_=s_):
                    pltpu.make_async_copy(a_hbm.at[pl.ds(0, GB)],
                                          gbufs[s_], sems[s_]).wait()
                    accum_block(g - (NRING - 1), gbufs[s_])
            return 0
        if not _PROBE_SKIP_GATHER:
            lax.fori_loop(0, ng + NRING - 1, pipe_body, 0)
        return 0

    lax.fori_loop(0, NCHUNK, chunk_body, 0)

    for k in range(RPW // GB):
        rbase = lo + k * GB
        pltpu.sync_copy(c_hbm.at[pl.ds(rbase, GB)], gb0)

        def comb_body(r, _):
            ra = k * GB + r
            for f in range(8):
                sl = pl.ds(f * 16, 16)
                a = acc[ra, sl]
                cv = gb0[r, sl]
                gb0[r, sl] = jnp.where(a < jnp.float32(_INF), cv - a,
                                       jnp.float32(0.0))
            return 0
        lax.fori_loop(0, GB, comb_body, 0)
        pltpu.sync_copy(gb0, out_hbm.at[pl.ds(rbase, GB)])


_sc_fn = pl.kernel(
    _sc_body,
    out_type=jax.ShapeDtypeStruct((NPAD, D), jnp.float32),
    mesh=plsc.VectorSubcoreMesh(core_axis_name="c", subcore_axis_name="s"),
    scratch_types=[
        pltpu.VMEM((RPW + 1, D), jnp.float32),
        pltpu.VMEM((GB, D), jnp.float32),
        pltpu.VMEM((GB, D), jnp.float32),
        pltpu.VMEM((GB, D), jnp.float32),
        pltpu.VMEM((GB, D), jnp.float32),
        pltpu.VMEM((CH,), jnp.int32),
        pltpu.VMEM((CH,), jnp.int32),
        pltpu.VMEM((CB,), jnp.int32),
        pltpu.VMEM((CB,), jnp.int32),
        pltpu.SemaphoreType.DMA,
        pltpu.SemaphoreType.DMA,
        pltpu.SemaphoreType.DMA,
        pltpu.SemaphoreType.DMA,
        pltpu.SemaphoreType.DMA,
        pltpu.SemaphoreType.DMA,
    ],
    compiler_params=pltpu.CompilerParams(needs_layout_passes=False),
)


@jax.jit
def kernel(x, edge_index, W_theta, b_theta, W_phi, b_phi):
    src = edge_index[0]
    dst = edge_index[1]
    xp = jnp.pad(x, ((0, NPAD - N), (0, 0)))
    a, c = _tc_fn(xp, W_theta, W_phi,
                  b_theta.reshape(1, D), b_phi.reshape(1, D))
    out = _sc_fn(a, c, src, dst)
    return out[:N]


# speedup vs baseline: 1.8715x; 1.2944x over previous
import functools

import jax
import jax.numpy as jnp
from jax import lax
from jax.experimental import pallas as pl
from jax.experimental.pallas import tpu as pltpu
from jax.experimental.pallas import tpu_sc as plsc

N = 10000
E = 320000
D = 128

NC = 2
NS = 16
NW = NC * NS
RPW = 320
NPAD = NW * RPW

CH = 16000
NCHUNK = E // CH
GRP = CH // 16
GB = 32
NRING = 4
TRASH = CH + GB
CB = TRASH + 16

_INF = float("inf")
_PROBE_SKIP_GATHER = False
_PROBE_SKIP_ACCUM = False


def _tc_body(x_ref, wt_ref, wp_ref, bt_ref, bp_ref, a_ref, c_ref):
    xb = x_ref[...]
    dn = (((1,), (1,)), ((), ()))
    a = lax.dot_general(xb, wt_ref[...], dn, preferred_element_type=jnp.float32)
    p = lax.dot_general(xb, wp_ref[...], dn, preferred_element_type=jnp.float32)
    a_ref[...] = a
    c_ref[...] = a + p + bt_ref[...] + bp_ref[...]


def _tc_fn(xp, wt, wp, bt, bp):
    grid = 8
    blk = NPAD // grid
    return pl.pallas_call(
        _tc_body,
        grid=(grid,),
        in_specs=[
            pl.BlockSpec((blk, D), lambda i: (i, 0)),
            pl.BlockSpec((D, D), lambda i: (0, 0)),
            pl.BlockSpec((D, D), lambda i: (0, 0)),
            pl.BlockSpec((1, D), lambda i: (0, 0)),
            pl.BlockSpec((1, D), lambda i: (0, 0)),
        ],
        out_specs=[
            pl.BlockSpec((blk, D), lambda i: (i, 0)),
            pl.BlockSpec((blk, D), lambda i: (i, 0)),
        ],
        out_shape=[
            jax.ShapeDtypeStruct((NPAD, D), jnp.float32),
            jax.ShapeDtypeStruct((NPAD, D), jnp.float32),
        ],
    )(xp, wt, wp, bt, bp)


def _sc_body(a_hbm, c_hbm, src_hbm, dst_hbm, out_hbm,
             acc, gb0, gb1, gb2, gb3, dst_v, src_v, scmp, lcmp,
             sm0, sm1, sm2, sm3, semd, semsrc):
    gbufs = (gb0, gb1, gb2, gb3)
    sems = (sm0, sm1, sm2, sm3)
    wid = lax.axis_index("s") * NC + lax.axis_index("c")
    lo = wid * RPW
    hi = lo + RPW

    inf_vec = jnp.full((16,), _INF, jnp.float32)
    zero_vec = jnp.zeros((16,), jnp.int32)
    ones16 = jnp.full((16,), 1, jnp.int32)
    dummy_vec = jnp.full((16,), RPW, jnp.int32)
    zeros16 = jnp.zeros((16,), jnp.int32)

    def init_row(r, _):
        for f in range(8):
            acc[r, pl.ds(f * 16, 16)] = inf_vec
        return 0
    lax.fori_loop(0, RPW + 1, init_row, 0)

    pltpu.async_copy(dst_hbm.at[pl.ds(0, CH)], dst_v, semd)
    pltpu.async_copy(src_hbm.at[pl.ds(0, CH)], src_v, semsrc)

    def chunk_body(ci, _):
        pltpu.make_async_copy(dst_hbm.at[pl.ds(0, CH)], dst_v, semd).wait()
        pltpu.make_async_copy(src_hbm.at[pl.ds(0, CH)], src_v, semsrc).wait()

        @plsc.parallel_loop(0, GRP, carry=jnp.int32(0), unroll=4)
        def scan_loop(g, cur):
            off = g * 16
            dvec = dst_v[pl.ds(off, 16)]
            svec = src_v[pl.ds(off, 16)]
            mask = jnp.logical_and(dvec >= lo, dvec < hi)
            cs = plsc.cumsum(jnp.where(mask, ones16, zeros16))
            lane = lax.iota(jnp.int32, 16)
            pos = jnp.where(mask, cur + cs - 1, TRASH + lane)
            plsc.store_scatter(scmp, [pos], svec)
            plsc.store_scatter(lcmp, [pos], dvec - lo)
            return cur + cs[15]
        n = scan_loop

        @pl.when(ci + 1 < NCHUNK)
        def _():
            nbase = (ci + 1) * CH
            pltpu.async_copy(dst_hbm.at[pl.ds(nbase, CH)], dst_v, semd)
            pltpu.async_copy(src_hbm.at[pl.ds(nbase, CH)], src_v, semsrc)

        def pad_body(t, _):
            scmp[pl.ds(n + t * 16, 16)] = zero_vec
            lcmp[pl.ds(n + t * 16, 16)] = dummy_vec
            return 0
        lax.fori_loop(0, GB // 16, pad_body, 0)

        ng = (n + GB - 1) // GB

        def accum_block(g, buf):
            base = g * GB

            def batch_body(t, _):
                lvec = lcmp[pl.ds(base + t * 16, 16)]
                for i_ in range(16):
                    r = lvec[i_]
                    j = t * 16 + i_
                    for f in range(8):
                        sl = pl.ds(f * 16, 16)
                        acc[r, sl] = jnp.minimum(acc[r, sl], buf[j, sl])
                return 0
            if not _PROBE_SKIP_ACCUM:
                lax.fori_loop(0, GB // 16, batch_body, 0)

        def pipe_body(g, _):
            for s_ in range(NRING):
                @pl.when(jnp.logical_and(g < ng, g % NRING == s_))
                def _(s_=s_):
                    pltpu.async_copy(a_hbm.at[scmp.at[pl.ds(g * GB, GB)]],
                                     gbufs[s_], sems[s_])

            for s_ in range(NRING):
                @pl.when(jnp.logical_and(g >= NRING - 1,
                                         (g - (NRING - 1)) % NRING == s_))
                def _(s_=s_):
                    pltpu.make_async_copy(a_hbm.at[pl.ds(0, GB)],
                                          gbufs[s_], sems[s_]).wait()
                    accum_block(g - (NRING - 1), gbufs[s_])
            return 0
        if not _PROBE_SKIP_GATHER:
            lax.fori_loop(0, ng + NRING - 1, pipe_body, 0)
        return 0

    lax.fori_loop(0, NCHUNK, chunk_body, 0)

    for k in range(RPW // GB):
        rbase = lo + k * GB
        pltpu.sync_copy(c_hbm.at[pl.ds(rbase, GB)], gb0)

        def comb_body(r, _):
            ra = k * GB + r
            for f in range(8):
                sl = pl.ds(f * 16, 16)
                a = acc[ra, sl]
                cv = gb0[r, sl]
                gb0[r, sl] = jnp.where(a < jnp.float32(_INF), cv - a,
                                       jnp.float32(0.0))
            return 0
        lax.fori_loop(0, GB, comb_body, 0)
        pltpu.sync_copy(gb0, out_hbm.at[pl.ds(rbase, GB)])


_sc_fn = pl.kernel(
    _sc_body,
    out_type=jax.ShapeDtypeStruct((NPAD, D), jnp.float32),
    mesh=plsc.VectorSubcoreMesh(core_axis_name="c", subcore_axis_name="s"),
    scratch_types=[
        pltpu.VMEM((RPW + 1, D), jnp.float32),
        pltpu.VMEM((GB, D), jnp.float32),
        pltpu.VMEM((GB, D), jnp.float32),
        pltpu.VMEM((GB, D), jnp.float32),
        pltpu.VMEM((GB, D), jnp.float32),
        pltpu.VMEM((CH,), jnp.int32),
        pltpu.VMEM((CH,), jnp.int32),
        pltpu.VMEM((CB,), jnp.int32),
        pltpu.VMEM((CB,), jnp.int32),
        pltpu.SemaphoreType.DMA,
        pltpu.SemaphoreType.DMA,
        pltpu.SemaphoreType.DMA,
        pltpu.SemaphoreType.DMA,
        pltpu.SemaphoreType.DMA,
        pltpu.SemaphoreType.DMA,
    ],
    compiler_params=pltpu.CompilerParams(needs_layout_passes=False),
)


@jax.jit
def kernel(x, edge_index, W_theta, b_theta, W_phi, b_phi):
    src = edge_index[0]
    dst = edge_index[1]
    xp = jnp.pad(x, ((0, NPAD - N), (0, 0)))
    a, c = _tc_fn(xp, W_theta, W_phi,
                  b_theta.reshape(1, D), b_phi.reshape(1, D))
    out = _sc_fn(a, c, src, dst)
    return out[:N]
